# Initial kernel scaffold; baseline (speedup 1.0000x reference)
import jax
import jax.numpy as jnp
from jax.experimental import pallas as pl


def kernel(x, edge_index, W1n, W1s, b1, g1, be1, W2n, W2s, b2, g2, be2, W3n, W3s, b3):
    raise NotImplementedError("write your pallas kernel here")


# speedup vs baseline: 4.7494x; 4.7494x over previous
import jax
import jax.numpy as jnp
from jax import lax
from jax.experimental import pallas as pl
from jax.experimental.pallas import tpu as pltpu
from jax.experimental.pallas import tpu_sc as plsc

_N = 10000
_E = 320000
_D = 128
_NC = 2
_NS = 16
_NW = _NC * _NS
_C = 80
_T = _E // _C
_TPW = _T // _NW
_RPT = (_N // _NS) // 8 * 8
_RTAIL = _N - _NS * _RPT
_SB = 208
_DW = 16

_mesh = plsc.VectorSubcoreMesh(core_axis_name="c", subcore_axis_name="s")


def _sc_agg_body(h_hbm, src_hbm, dst_hbm, zf_hbm, agg_out,
                 src_i, dst_i, rows, sbuf, agg_sh, sem):
    c = lax.axis_index("c")
    s = lax.axis_index("s")
    wid = s * _NC + c
    tb = _NS * _RPT

    pltpu.sync_copy(zf_hbm, sbuf)
    for k in range(_RPT // _SB):
        pltpu.sync_copy(sbuf, agg_sh.at[pl.ds(s * _RPT + k * _SB, _SB)])

    @pl.when(s == 0)
    def _():
        pltpu.sync_copy(sbuf.at[pl.ds(0, _RTAIL)], agg_sh.at[pl.ds(tb, _RTAIL)])

    plsc.subcore_barrier()

    def step(i, carry):
        base = (wid + i * _NW) * _C
        pltpu.sync_copy(src_hbm.at[pl.ds(base, _C)], src_i)
        pltpu.sync_copy(dst_hbm.at[pl.ds(base, _C)], dst_i)
        pltpu.async_copy(h_hbm.at[src_i], rows, sem).wait()
        pltpu.sync_copy(rows, agg_sh.at[dst_i], add=True)
        return carry

    lax.fori_loop(0, _TPW, step, 0)
    plsc.subcore_barrier()

    for k in range(_RPT // _SB):
        off = s * _RPT + k * _SB
        pltpu.sync_copy(agg_sh.at[pl.ds(off, _SB)], sbuf)
        pltpu.sync_copy(sbuf, agg_out.at[c].at[pl.ds(off, _SB)])

    @pl.when(s == 0)
    def _():
        pltpu.sync_copy(agg_sh.at[pl.ds(tb, _RTAIL)], sbuf.at[pl.ds(0, _RTAIL)])
        pltpu.sync_copy(sbuf.at[pl.ds(0, _RTAIL)],
                        agg_out.at[c].at[pl.ds(tb, _RTAIL)])


_sc_agg = pl.kernel(
    _sc_agg_body,
    out_type=[jax.ShapeDtypeStruct((_NC, _N, _D), jnp.float32)],
    mesh=_mesh,
    scratch_types=[
        pltpu.VMEM((_C,), jnp.int32),
        pltpu.VMEM((_C,), jnp.int32),
        pltpu.VMEM((_C, _D), jnp.float32),
        pltpu.VMEM((_SB, _D), jnp.float32),
        pltpu.VMEM_SHARED((_N, _D), jnp.float32),
        pltpu.SemaphoreType.DMA,
    ],
)


def _sc_deg_body(dst_hbm, zf_hbm, ones_hbm, deg_out,
                 dst_i, ones_v, sbuf, deg_sh, sem):
    c = lax.axis_index("c")
    s = lax.axis_index("s")
    wid = s * _NC + c
    tb = _NS * _RPT

    pltpu.sync_copy(zf_hbm, sbuf)
    for k in range(_RPT // _SB):
        pltpu.sync_copy(sbuf, deg_sh.at[pl.ds(s * _RPT + k * _SB, _SB)])
    pltpu.sync_copy(ones_hbm, ones_v)

    @pl.when(s == 0)
    def _():
        pltpu.sync_copy(sbuf.at[pl.ds(0, _RTAIL)], deg_sh.at[pl.ds(tb, _RTAIL)])

    plsc.subcore_barrier()

    def step(i, carry):
        base = (wid + i * _NW) * _C
        pltpu.sync_copy(dst_hbm.at[pl.ds(base, _C)], dst_i)
        pltpu.sync_copy(ones_v, deg_sh.at[dst_i], add=True)
        return carry

    lax.fori_loop(0, _TPW, step, 0)
    plsc.subcore_barrier()

    for k in range(_RPT // _SB):
        off = s * _RPT + k * _SB
        pltpu.sync_copy(deg_sh.at[pl.ds(off, _SB)], sbuf)
        pltpu.sync_copy(sbuf, deg_out.at[c].at[pl.ds(off, _SB)])

    @pl.when(s == 0)
    def _():
        pltpu.sync_copy(deg_sh.at[pl.ds(tb, _RTAIL)], sbuf.at[pl.ds(0, _RTAIL)])
        pltpu.sync_copy(sbuf.at[pl.ds(0, _RTAIL)],
                        deg_out.at[c].at[pl.ds(tb, _RTAIL)])


_sc_deg = pl.kernel(
    _sc_deg_body,
    out_type=[jax.ShapeDtypeStruct((_NC, _N, _D), jnp.float32)],
    mesh=_mesh,
    scratch_types=[
        pltpu.VMEM((_C,), jnp.int32),
        pltpu.VMEM((_C, _D), jnp.float32),
        pltpu.VMEM((_SB, _D), jnp.float32),
        pltpu.VMEM_SHARED((_N, _D), jnp.float32),
        pltpu.SemaphoreType.DMA,
    ],
)


def _mm_t(a, w):
    return lax.dot_general(a, w, (((1,), (1,)), ((), ())),
                           preferred_element_type=jnp.float32)


def _tc_bn_body(agg_ref, deg_ref, h_ref, wn_ref, ws_ref, b_ref, g_ref,
                be_ref, out_ref):
    dd = deg_ref[0] + deg_ref[1]
    invd = 1.0 / jnp.maximum(dd[:, 0:1], 1.0)
    agg = (agg_ref[0] + agg_ref[1]) * invd
    y = _mm_t(agg, wn_ref[...]) + _mm_t(h_ref[...], ws_ref[...]) + b_ref[...]
    m = jnp.mean(y, axis=0, keepdims=True)
    v = jnp.mean((y - m) ** 2, axis=0, keepdims=True)
    yn = (y - m) * lax.rsqrt(v + 1e-5) * g_ref[...] + be_ref[...]
    out_ref[...] = jnp.where(yn >= 0, yn, 0.1 * yn)


def _tc_final_body(agg_ref, deg_ref, h_ref, wn_ref, ws_ref, b_ref, out_ref):
    dd = deg_ref[0] + deg_ref[1]
    invd = 1.0 / jnp.maximum(dd[:, 0:1], 1.0)
    agg = (agg_ref[0] + agg_ref[1]) * invd
    out_ref[...] = (_mm_t(agg, wn_ref[...]) + _mm_t(h_ref[...], ws_ref[...])
                    + b_ref[...])


def _tc_bn(agg, deg, h, wn, ws, b, g, be):
    return pl.pallas_call(
        _tc_bn_body,
        out_shape=jax.ShapeDtypeStruct((_N, _D), jnp.float32),
    )(agg, deg, h, wn, ws, b, g, be)


def _tc_final(agg, deg, h, wn, ws, b):
    return pl.pallas_call(
        _tc_final_body,
        out_shape=jax.ShapeDtypeStruct((_N, _D), jnp.float32),
    )(agg, deg, h, wn, ws, b)


def kernel(x, edge_index, W1n, W1s, b1, g1, be1, W2n, W2s, b2, g2, be2,
           W3n, W3s, b3):
    src = edge_index[0].astype(jnp.int32)
    dst = edge_index[1].astype(jnp.int32)
    zf = jnp.zeros((_SB, _D), jnp.float32)
    ones = jnp.ones((_C, _D), jnp.float32)
    b1r, g1r, be1r = b1.reshape(1, -1), g1.reshape(1, -1), be1.reshape(1, -1)
    b2r, g2r, be2r = b2.reshape(1, -1), g2.reshape(1, -1), be2.reshape(1, -1)
    b3r = b3.reshape(1, -1)

    (deg,) = _sc_deg(dst, zf, ones)
    (agg1,) = _sc_agg(x, src, dst, zf)
    h1 = _tc_bn(agg1, deg, x, W1n, W1s, b1r, g1r, be1r)
    (agg2,) = _sc_agg(h1, src, dst, zf)
    h2 = _tc_bn(agg2, deg, h1, W2n, W2s, b2r, g2r, be2r)
    (agg3,) = _sc_agg(h2, src, dst, zf)
    return _tc_final(agg3, deg, h2, W3n, W3s, b3r)


# speedup vs baseline: 7.1401x; 1.5034x over previous
import jax
import jax.numpy as jnp
from jax import lax
from jax.experimental import pallas as pl
from jax.experimental.pallas import tpu as pltpu
from jax.experimental.pallas import tpu_sc as plsc

_N = 10000
_E = 320000
_D = 128
_NC = 2
_NS = 16
_NW = _NC * _NS
_C = 80
_T = _E // _C
_TPW = _T // _NW
_RPT = (_N // _NS) // 8 * 8
_RTAIL = _N - _NS * _RPT
_SB = 208
_DW = 16

_mesh = plsc.VectorSubcoreMesh(core_axis_name="c", subcore_axis_name="s")


def _sc_agg_body(h_hbm, src_hbm, dst_hbm, zf_hbm, agg_out,
                 src_i, dst_i, rows, sbuf, agg_sh, gsem0, gsem1):
    c = lax.axis_index("c")
    s = lax.axis_index("s")
    wid = s * _NC + c
    wbase = wid * (_E // _NW)
    tb = _NS * _RPT

    pltpu.sync_copy(zf_hbm, sbuf)
    for k in range(_RPT // _SB):
        pltpu.sync_copy(sbuf, agg_sh.at[pl.ds(s * _RPT + k * _SB, _SB)])

    @pl.when(s == 0)
    def _():
        pltpu.sync_copy(sbuf.at[pl.ds(0, _RTAIL)], agg_sh.at[pl.ds(tb, _RTAIL)])

    pltpu.sync_copy(src_hbm.at[pl.ds(wbase, _C)], src_i.at[0])
    pltpu.sync_copy(dst_hbm.at[pl.ds(wbase, _C)], dst_i.at[0])
    pltpu.async_copy(h_hbm.at[src_i.at[0]], rows.at[0], gsem0)

    plsc.subcore_barrier()

    sems = (gsem0, gsem1)

    def pair(j, carry):
        for b in range(2):
            i = 2 * j + b
            nb = 1 - b
            nbase = wbase + (i + 1) * _C
            pltpu.sync_copy(src_hbm.at[pl.ds(nbase, _C)], src_i.at[nb])
            pltpu.sync_copy(dst_hbm.at[pl.ds(nbase, _C)], dst_i.at[nb])
            pltpu.async_copy(h_hbm.at[src_i.at[nb]], rows.at[nb], sems[nb])
            pltpu.make_async_copy(h_hbm.at[src_i.at[b]], rows.at[b],
                                  sems[b]).wait()
            pltpu.sync_copy(rows.at[b], agg_sh.at[dst_i.at[b]], add=True)
        return carry

    lax.fori_loop(0, (_TPW - 1) // 2, pair, 0)
    pltpu.make_async_copy(h_hbm.at[src_i.at[0]], rows.at[0], gsem0).wait()
    pltpu.sync_copy(rows.at[0], agg_sh.at[dst_i.at[0]], add=True)
    plsc.subcore_barrier()

    for k in range(_RPT // _SB):
        off = s * _RPT + k * _SB
        pltpu.sync_copy(agg_sh.at[pl.ds(off, _SB)], sbuf)
        pltpu.sync_copy(sbuf, agg_out.at[c].at[pl.ds(off, _SB)])

    @pl.when(s == 0)
    def _():
        pltpu.sync_copy(agg_sh.at[pl.ds(tb, _RTAIL)], sbuf.at[pl.ds(0, _RTAIL)])
        pltpu.sync_copy(sbuf.at[pl.ds(0, _RTAIL)],
                        agg_out.at[c].at[pl.ds(tb, _RTAIL)])


_sc_agg = pl.kernel(
    _sc_agg_body,
    out_type=[jax.ShapeDtypeStruct((_NC, _N, _D), jnp.float32)],
    mesh=_mesh,
    scratch_types=[
        pltpu.VMEM((2, _C), jnp.int32),
        pltpu.VMEM((2, _C), jnp.int32),
        pltpu.VMEM((2, _C, _D), jnp.float32),
        pltpu.VMEM((_SB, _D), jnp.float32),
        pltpu.VMEM_SHARED((_N, _D), jnp.float32),
        pltpu.SemaphoreType.DMA,
        pltpu.SemaphoreType.DMA,
    ],
)


def _sc_deg_body(dst_hbm, zf_hbm, ones_hbm, deg_out,
                 dst_i, ones_v, sbuf, deg_sh, sem):
    c = lax.axis_index("c")
    s = lax.axis_index("s")
    wid = s * _NC + c
    tb = _NS * _RPT

    pltpu.sync_copy(zf_hbm, sbuf)
    for k in range(_RPT // _SB):
        pltpu.sync_copy(sbuf, deg_sh.at[pl.ds(s * _RPT + k * _SB, _SB)])
    pltpu.sync_copy(ones_hbm, ones_v)

    @pl.when(s == 0)
    def _():
        pltpu.sync_copy(sbuf.at[pl.ds(0, _RTAIL)], deg_sh.at[pl.ds(tb, _RTAIL)])

    plsc.subcore_barrier()

    def step(i, carry):
        base = (wid + i * _NW) * _C
        pltpu.sync_copy(dst_hbm.at[pl.ds(base, _C)], dst_i)
        pltpu.sync_copy(ones_v, deg_sh.at[dst_i], add=True)
        return carry

    lax.fori_loop(0, _TPW, step, 0)
    plsc.subcore_barrier()

    for k in range(_RPT // _SB):
        off = s * _RPT + k * _SB
        pltpu.sync_copy(deg_sh.at[pl.ds(off, _SB)], sbuf)
        pltpu.sync_copy(sbuf, deg_out.at[c].at[pl.ds(off, _SB)])

    @pl.when(s == 0)
    def _():
        pltpu.sync_copy(deg_sh.at[pl.ds(tb, _RTAIL)], sbuf.at[pl.ds(0, _RTAIL)])
        pltpu.sync_copy(sbuf.at[pl.ds(0, _RTAIL)],
                        deg_out.at[c].at[pl.ds(tb, _RTAIL)])


_sc_deg = pl.kernel(
    _sc_deg_body,
    out_type=[jax.ShapeDtypeStruct((_NC, _N, _D), jnp.float32)],
    mesh=_mesh,
    scratch_types=[
        pltpu.VMEM((_C,), jnp.int32),
        pltpu.VMEM((_C, _D), jnp.float32),
        pltpu.VMEM((_SB, _D), jnp.float32),
        pltpu.VMEM_SHARED((_N, _D), jnp.float32),
        pltpu.SemaphoreType.DMA,
    ],
)


def _mm_t(a, w):
    return lax.dot_general(a, w, (((1,), (1,)), ((), ())),
                           preferred_element_type=jnp.float32)


def _tc_bn_body(agg_ref, deg_ref, h_ref, wn_ref, ws_ref, b_ref, g_ref,
                be_ref, out_ref):
    dd = deg_ref[0] + deg_ref[1]
    invd = 1.0 / jnp.maximum(dd[:, 0:1], 1.0)
    agg = (agg_ref[0] + agg_ref[1]) * invd
    y = _mm_t(agg, wn_ref[...]) + _mm_t(h_ref[...], ws_ref[...]) + b_ref[...]
    m = jnp.mean(y, axis=0, keepdims=True)
    v = jnp.mean((y - m) ** 2, axis=0, keepdims=True)
    yn = (y - m) * lax.rsqrt(v + 1e-5) * g_ref[...] + be_ref[...]
    out_ref[...] = jnp.where(yn >= 0, yn, 0.1 * yn)


def _tc_final_body(agg_ref, deg_ref, h_ref, wn_ref, ws_ref, b_ref, out_ref):
    dd = deg_ref[0] + deg_ref[1]
    invd = 1.0 / jnp.maximum(dd[:, 0:1], 1.0)
    agg = (agg_ref[0] + agg_ref[1]) * invd
    out_ref[...] = (_mm_t(agg, wn_ref[...]) + _mm_t(h_ref[...], ws_ref[...])
                    + b_ref[...])


def _tc_bn(agg, deg, h, wn, ws, b, g, be):
    return pl.pallas_call(
        _tc_bn_body,
        out_shape=jax.ShapeDtypeStruct((_N, _D), jnp.float32),
    )(agg, deg, h, wn, ws, b, g, be)


def _tc_final(agg, deg, h, wn, ws, b):
    return pl.pallas_call(
        _tc_final_body,
        out_shape=jax.ShapeDtypeStruct((_N, _D), jnp.float32),
    )(agg, deg, h, wn, ws, b)


def kernel(x, edge_index, W1n, W1s, b1, g1, be1, W2n, W2s, b2, g2, be2,
           W3n, W3s, b3):
    src = edge_index[0].astype(jnp.int32)
    dst = edge_index[1].astype(jnp.int32)
    zf = jnp.zeros((_SB, _D), jnp.float32)
    ones = jnp.ones((_C, _D), jnp.float32)
    b1r, g1r, be1r = b1.reshape(1, -1), g1.reshape(1, -1), be1.reshape(1, -1)
    b2r, g2r, be2r = b2.reshape(1, -1), g2.reshape(1, -1), be2.reshape(1, -1)
    b3r = b3.reshape(1, -1)

    (deg,) = _sc_deg(dst, zf, ones)
    (agg1,) = _sc_agg(x, src, dst, zf)
    h1 = _tc_bn(agg1, deg, x, W1n, W1s, b1r, g1r, be1r)
    (agg2,) = _sc_agg(h1, src, dst, zf)
    h2 = _tc_bn(agg2, deg, h1, W2n, W2s, b2r, g2r, be2r)
    (agg3,) = _sc_agg(h2, src, dst, zf)
    return _tc_final(agg3, deg, h2, W3n, W3s, b3r)
